# Initial kernel scaffold; baseline (speedup 1.0000x reference)
import jax
import jax.numpy as jnp
from jax.experimental import pallas as pl


def kernel(x_user, x_movie, edge_label_index):
    raise NotImplementedError("write your pallas kernel here")


# speedup vs baseline: 1.0527x; 1.0527x over previous
import functools

import jax
import jax.numpy as jnp
from jax import lax
from jax.experimental import pallas as pl
from jax.experimental.pallas import tpu as pltpu
from jax.experimental.pallas import tpu_sc as plsc

L = 16
NC = 2
NS = 16
NW = NC * NS
CHUNK = 128
SUPER = 8192


def _make_sc_kernel(n_user, n_movie, dim, e):
    assert dim % L == 0 and e % (NW * SUPER) == 0
    epw = e // NW
    nch = SUPER // CHUNK
    nsuper = epw // SUPER
    mesh = plsc.VectorSubcoreMesh(core_axis_name="c", subcore_axis_name="s")

    @functools.partial(
        pl.kernel,
        mesh=mesh,
        compiler_params=pltpu.CompilerParams(
            needs_layout_passes=False, use_tc_tiling_on_sc=False),
        out_type=jax.ShapeDtypeStruct((e,), jnp.float32),
        scratch_types=[
            pltpu.VMEM((nch, CHUNK), jnp.int32),
            pltpu.VMEM((nch, CHUNK), jnp.int32),
            pltpu.VMEM((CHUNK, dim), jnp.float32),
            pltpu.VMEM((CHUNK, dim), jnp.float32),
            pltpu.VMEM((epw,), jnp.float32),
            pltpu.SemaphoreType.DMA,
            pltpu.SemaphoreType.DMA,
        ],
    )
    def sc_kernel(xu, xm, uix, mix, out, uidx_v, midx_v, ru, rm,
                  out_v, su, sm):
        cid = lax.axis_index("c")
        sid = lax.axis_index("s")
        wid = sid * NC + cid
        base = wid * epw
        iota16 = lax.iota(jnp.int32, L)

        def compute_chunk(out_off):
            def group_body(g, carry):
                res = jnp.zeros((L,), jnp.float32)
                for el in range(L):
                    row = g * L + el
                    acc = ru[row, pl.ds(0, L)] * rm[row, pl.ds(0, L)]
                    for k in range(1, dim // L):
                        acc = acc + (ru[row, pl.ds(k * L, L)]
                                     * rm[row, pl.ds(k * L, L)])
                    res = jnp.where(iota16 == el, jnp.sum(acc), res)
                pred = 1.0 / (1.0 + jnp.exp(-res))
                out_v[pl.ds(out_off + g * L, L)] = pred
                return carry
            lax.fori_loop(0, CHUNK // L, group_body, 0)

        def super_body(s, carry):
            srow = pl.multiple_of((base // CHUNK) + s * nch, nch)
            pltpu.sync_copy(uix.at[pl.ds(srow, nch)], uidx_v)
            pltpu.sync_copy(mix.at[pl.ds(srow, nch)], midx_v)

            def chunk_body(j, c2):
                cu = pltpu.async_copy(xu.at[uidx_v.at[j]], ru, su)
                cm = pltpu.async_copy(xm.at[midx_v.at[j]], rm, sm)
                cu.wait()
                cm.wait()
                compute_chunk(s * SUPER + j * CHUNK)
                return c2
            lax.fori_loop(0, nch, chunk_body, 0)
            return carry

        lax.fori_loop(0, nsuper, super_body, 0)
        pltpu.sync_copy(out_v, out.at[pl.ds(base, epw)])

    return sc_kernel


def kernel(x_user, x_movie, edge_label_index):
    n_user, dim = x_user.shape
    n_movie, _ = x_movie.shape
    e = edge_label_index.shape[1]
    eli = edge_label_index.astype(jnp.int32)
    uix = eli[0].reshape(e // CHUNK, CHUNK)
    mix = eli[1].reshape(e // CHUNK, CHUNK)
    sc = _make_sc_kernel(n_user, n_movie, dim, e)
    return sc(x_user, x_movie, uix, mix)


# speedup vs baseline: 1.0535x; 1.0008x over previous
import functools

import jax
import jax.numpy as jnp
from jax import lax
from jax.experimental import pallas as pl
from jax.experimental.pallas import tpu as pltpu
from jax.experimental.pallas import tpu_sc as plsc

L = 16
NC = 2
NS = 16
NW = NC * NS
CHUNK = 128
SUPER = 8192


def _make_sc_kernel(n_user, n_movie, dim, e):
    assert dim % L == 0 and e % (NW * SUPER) == 0
    epw = e // NW
    nch = SUPER // CHUNK
    nsuper = epw // SUPER
    mesh = plsc.VectorSubcoreMesh(core_axis_name="c", subcore_axis_name="s")

    @functools.partial(
        pl.kernel,
        mesh=mesh,
        compiler_params=pltpu.CompilerParams(
            needs_layout_passes=False, use_tc_tiling_on_sc=False),
        out_type=jax.ShapeDtypeStruct((e,), jnp.float32),
        scratch_types=[
            pltpu.VMEM((SUPER,), jnp.int32),
            pltpu.VMEM((SUPER,), jnp.int32),
            pltpu.VMEM((CHUNK, dim), jnp.float32),
            pltpu.VMEM((CHUNK, dim), jnp.float32),
            pltpu.VMEM((epw,), jnp.float32),
            pltpu.SemaphoreType.DMA,
            pltpu.SemaphoreType.DMA,
        ],
    )
    def sc_kernel(xu, xm, eli, out, uidx_v, midx_v, ru, rm,
                  out_v, su, sm):
        cid = lax.axis_index("c")
        sid = lax.axis_index("s")
        wid = sid * NC + cid
        base = wid * epw
        iota16 = lax.iota(jnp.int32, L)

        def compute_chunk(out_off):
            def group_body(g, carry):
                res = jnp.zeros((L,), jnp.float32)
                for el in range(L):
                    row = g * L + el
                    acc = ru[row, pl.ds(0, L)] * rm[row, pl.ds(0, L)]
                    for k in range(1, dim // L):
                        acc = acc + (ru[row, pl.ds(k * L, L)]
                                     * rm[row, pl.ds(k * L, L)])
                    res = jnp.where(iota16 == el, jnp.sum(acc), res)
                pred = 1.0 / (1.0 + jnp.exp(-res))
                out_v[pl.ds(out_off + g * L, L)] = pred
                return carry
            lax.fori_loop(0, CHUNK // L, group_body, 0)

        def super_body(s, carry):
            soff = pl.multiple_of(base + s * SUPER, SUPER)
            pltpu.sync_copy(eli.at[0, pl.ds(soff, SUPER)], uidx_v)
            pltpu.sync_copy(eli.at[1, pl.ds(soff, SUPER)], midx_v)

            def chunk_body(j, c2):
                jj = pl.multiple_of(j * CHUNK, CHUNK)
                cu = pltpu.async_copy(
                    xu.at[uidx_v.at[pl.ds(jj, CHUNK)]], ru, su)
                cm = pltpu.async_copy(
                    xm.at[midx_v.at[pl.ds(jj, CHUNK)]], rm, sm)
                cu.wait()
                cm.wait()
                compute_chunk(s * SUPER + j * CHUNK)
                return c2
            lax.fori_loop(0, nch, chunk_body, 0)
            return carry

        lax.fori_loop(0, nsuper, super_body, 0)
        pltpu.sync_copy(out_v, out.at[pl.ds(base, epw)])

    return sc_kernel


def kernel(x_user, x_movie, edge_label_index):
    n_user, dim = x_user.shape
    n_movie, _ = x_movie.shape
    e = edge_label_index.shape[1]
    eli = edge_label_index.astype(jnp.int32)
    sc = _make_sc_kernel(n_user, n_movie, dim, e)
    return sc(x_user, x_movie, eli)


# speedup vs baseline: 1.2419x; 1.1788x over previous
import functools

import jax
import jax.numpy as jnp
from jax import lax
from jax.experimental import pallas as pl
from jax.experimental.pallas import tpu as pltpu
from jax.experimental.pallas import tpu_sc as plsc

L = 16
NC = 2
NS = 16
NW = NC * NS
CHUNK = 128
SUPER = 8192


def _make_sc_kernel(n_user, n_movie, dim, e):
    assert dim % L == 0 and e % (NW * SUPER) == 0
    epw = e // NW
    nch = SUPER // CHUNK
    nsuper = epw // SUPER
    mesh = plsc.VectorSubcoreMesh(core_axis_name="c", subcore_axis_name="s")

    @functools.partial(
        pl.kernel,
        mesh=mesh,
        compiler_params=pltpu.CompilerParams(
            needs_layout_passes=False, use_tc_tiling_on_sc=False),
        out_type=jax.ShapeDtypeStruct((e,), jnp.float32),
        scratch_types=[
            pltpu.VMEM((SUPER,), jnp.int32),
            pltpu.VMEM((SUPER,), jnp.int32),
            pltpu.VMEM((CHUNK, dim), jnp.float32),
            pltpu.VMEM((CHUNK, dim), jnp.float32),
            pltpu.VMEM((CHUNK, dim), jnp.float32),
            pltpu.VMEM((CHUNK, dim), jnp.float32),
            pltpu.VMEM((epw,), jnp.float32),
            pltpu.SemaphoreType.DMA,
            pltpu.SemaphoreType.DMA,
            pltpu.SemaphoreType.DMA,
            pltpu.SemaphoreType.DMA,
        ],
    )
    def sc_kernel(xu, xm, eli, out, uidx_v, midx_v, ru0, ru1, rm0, rm1,
                  out_v, su0, su1, sm0, sm1):
        cid = lax.axis_index("c")
        sid = lax.axis_index("s")
        wid = sid * NC + cid
        base = wid * epw
        iota16 = lax.iota(jnp.int32, L)
        rbufs = ((ru0, rm0, su0, sm0), (ru1, rm1, su1, sm1))

        def fire(b, j):
            ru, rm, su, sm = rbufs[b]
            jj = pl.multiple_of(j * CHUNK, CHUNK)
            pltpu.async_copy(xu.at[uidx_v.at[pl.ds(jj, CHUNK)]], ru, su)
            pltpu.async_copy(xm.at[midx_v.at[pl.ds(jj, CHUNK)]], rm, sm)

        def wait(b):
            ru, rm, su, sm = rbufs[b]
            pltpu.make_async_copy(xu.at[pl.ds(0, CHUNK)], ru, su).wait()
            pltpu.make_async_copy(xm.at[pl.ds(0, CHUNK)], rm, sm).wait()

        def compute_chunk(b, out_off):
            ru, rm, _, _ = rbufs[b]

            def group_body(g, carry):
                res = jnp.zeros((L,), jnp.float32)
                for el in range(L):
                    row = g * L + el
                    acc = ru[row, pl.ds(0, L)] * rm[row, pl.ds(0, L)]
                    for k in range(1, dim // L):
                        acc = acc + (ru[row, pl.ds(k * L, L)]
                                     * rm[row, pl.ds(k * L, L)])
                    res = jnp.where(iota16 == el, jnp.sum(acc), res)
                pred = 1.0 / (1.0 + jnp.exp(-res))
                out_v[pl.ds(out_off + g * L, L)] = pred
                return carry
            lax.fori_loop(0, CHUNK // L, group_body, 0)

        def super_body(s, carry):
            soff = pl.multiple_of(base + s * SUPER, SUPER)
            pltpu.sync_copy(eli.at[0, pl.ds(soff, SUPER)], uidx_v)
            pltpu.sync_copy(eli.at[1, pl.ds(soff, SUPER)], midx_v)

            fire(0, 0)

            def pair_body(jp, c2):
                j0 = jp * 2
                fire(1, j0 + 1)
                wait(0)
                compute_chunk(0, s * SUPER + j0 * CHUNK)

                @pl.when(j0 + 2 < nch)
                def _prefetch():
                    fire(0, j0 + 2)

                wait(1)
                compute_chunk(1, s * SUPER + (j0 + 1) * CHUNK)
                return c2
            lax.fori_loop(0, nch // 2, pair_body, 0)
            return carry

        lax.fori_loop(0, nsuper, super_body, 0)
        pltpu.sync_copy(out_v, out.at[pl.ds(base, epw)])

    return sc_kernel


def kernel(x_user, x_movie, edge_label_index):
    n_user, dim = x_user.shape
    n_movie, _ = x_movie.shape
    e = edge_label_index.shape[1]
    eli = edge_label_index.astype(jnp.int32)
    sc = _make_sc_kernel(n_user, n_movie, dim, e)
    return sc(x_user, x_movie, eli)
